# Initial kernel scaffold; baseline (speedup 1.0000x reference)
import jax
import jax.numpy as jnp
from jax.experimental import pallas as pl


def kernel(x, tables, W, b, lin_w, bias):
    raise NotImplementedError("write your pallas kernel here")


# speedup vs baseline: 7.2411x; 7.2411x over previous
import functools

import jax
import jax.numpy as jnp
from jax import lax
from jax.experimental import pallas as pl
from jax.experimental.pallas import tpu as pltpu
from jax.experimental.pallas import tpu_sc as plsc


def _build_sc_gather(N, D, n_chunks=4):
    NC, NS = 2, 16
    NW = NC * NS
    rows_per_worker = N // NW
    CH = rows_per_worker // n_chunks
    mesh = plsc.VectorSubcoreMesh(core_axis_name="c", subcore_axis_name="s")

    @functools.partial(
        pl.kernel,
        out_type=(
            jax.ShapeDtypeStruct((N, D), jnp.float32),
            jax.ShapeDtypeStruct((N,), jnp.float32),
        ),
        mesh=mesh,
        compiler_params=pltpu.CompilerParams(use_tc_tiling_on_sc=False),
        scratch_types=(
            pltpu.VMEM((CH,), jnp.int32),
            pltpu.VMEM((CH, D), jnp.float32),
            pltpu.VMEM((CH,), jnp.float32),
            pltpu.SemaphoreType.DMA,
            pltpu.SemaphoreType.DMA,
        ),
    )
    def gather_k(idx_hbm, tab_hbm, lw_hbm, e_out, lv_out,
                 idx_v, rows_v, lv_v, sem_r, sem_l):
        wid = lax.axis_index("s") * NC + lax.axis_index("c")
        base0 = wid * rows_per_worker
        for i in range(n_chunks):
            base = base0 + i * CH
            pltpu.sync_copy(idx_hbm.at[pl.ds(base, CH)], idx_v)
            cr = pltpu.async_copy(tab_hbm.at[idx_v], rows_v, sem_r)
            cl = pltpu.async_copy(lw_hbm.at[idx_v], lv_v, sem_l)
            cr.wait()
            cl.wait()
            pltpu.sync_copy(rows_v, e_out.at[pl.ds(base, CH)])
            pltpu.sync_copy(lv_v, lv_out.at[pl.ds(base, CH)])

    return gather_k


def _fm_body(e_ref, lv_ref, wblk_ref, bcat_ref, smat_ref, bias_ref, o_ref):
    E = e_ref[...]
    P = jnp.dot(E, wblk_ref[...],
                preferred_element_type=jnp.float32,
                precision=lax.Precision.HIGHEST) + bcat_ref[...]
    s = jnp.dot(P, smat_ref[...],
                preferred_element_type=jnp.float32,
                precision=lax.Precision.HIGHEST)
    ix = 0.5 * (jnp.sum(s * s, axis=1) - jnp.sum(P * P, axis=1))
    lin = jnp.sum(lv_ref[...], axis=1) + bias_ref[0]
    o_ref[...] = jax.nn.sigmoid(lin + ix)


def kernel(x, tables, W, b, lin_w, bias):
    F, V, D = tables.shape
    B = x.shape[0]
    N = B * F
    FD = F * D

    offs = (jnp.arange(F, dtype=x.dtype) * V)[None, :]
    flat_idx = (x + offs).reshape(-1)
    tab2 = tables.reshape(F * V, D)
    lw1 = lin_w.reshape(-1)
    rf = jnp.arange(F)
    wblk = (jnp.zeros((F, D, F, D), jnp.float32)
            .at[rf, :, rf, :].set(W.astype(jnp.float32))
            .reshape(FD, FD))
    bcat = b.astype(jnp.float32).reshape(1, FD)
    smat = jnp.tile(jnp.eye(D, dtype=jnp.float32), (F, 1))
    bias1 = jnp.asarray(bias, jnp.float32).reshape(1)

    e_flat, lv = _build_sc_gather(N, D)(flat_idx, tab2, lw1)
    e2 = e_flat.reshape(B, FD)
    lv2 = lv.reshape(B, F)

    BT = 2048
    out = pl.pallas_call(
        _fm_body,
        grid=(B // BT,),
        in_specs=[
            pl.BlockSpec((BT, FD), lambda i: (i, 0)),
            pl.BlockSpec((BT, F), lambda i: (i, 0)),
            pl.BlockSpec((FD, FD), lambda i: (0, 0)),
            pl.BlockSpec((1, FD), lambda i: (0, 0)),
            pl.BlockSpec((FD, D), lambda i: (0, 0)),
            pl.BlockSpec(memory_space=pltpu.SMEM),
        ],
        out_specs=pl.BlockSpec((BT,), lambda i: (i,)),
        out_shape=jax.ShapeDtypeStruct((B,), jnp.float32),
    )(e2, lv2, wblk, bcat, smat, bias1)
    return out


# speedup vs baseline: 7.5670x; 1.0450x over previous
import functools

import jax
import jax.numpy as jnp
from jax import lax
from jax.experimental import pallas as pl
from jax.experimental.pallas import tpu as pltpu
from jax.experimental.pallas import tpu_sc as plsc


def _build_sc_gather(N, D, n_chunks=4):
    NC, NS = 2, 16
    NW = NC * NS
    rows_per_worker = N // NW
    CH = rows_per_worker // n_chunks
    mesh = plsc.VectorSubcoreMesh(core_axis_name="c", subcore_axis_name="s")

    @functools.partial(
        pl.kernel,
        out_type=(
            jax.ShapeDtypeStruct((N, D), jnp.float32),
            jax.ShapeDtypeStruct((N,), jnp.float32),
        ),
        mesh=mesh,
        compiler_params=pltpu.CompilerParams(use_tc_tiling_on_sc=False),
        scratch_types=(
            pltpu.VMEM((CH,), jnp.int32),
            pltpu.VMEM((CH,), jnp.int32),
            pltpu.VMEM((CH, D), jnp.float32),
            pltpu.VMEM((CH,), jnp.float32),
            pltpu.SemaphoreType.DMA,
            pltpu.SemaphoreType.DMA,
        ),
    )
    def gather_k(idxe_hbm, idxl_hbm, tab_hbm, lw_hbm, e_out, lv_out,
                 idx_v, idxl_v, rows_v, lv_v, sem_r, sem_l):
        wid = lax.axis_index("s") * NC + lax.axis_index("c")
        base0 = wid * rows_per_worker
        for i in range(n_chunks):
            base = base0 + i * CH
            pltpu.sync_copy(idxe_hbm.at[pl.ds(base, CH)], idx_v)
            pltpu.sync_copy(idxl_hbm.at[pl.ds(base, CH)], idxl_v)
            cr = pltpu.async_copy(tab_hbm.at[idx_v], rows_v, sem_r)
            cl = pltpu.async_copy(lw_hbm.at[idxl_v], lv_v, sem_l)
            cr.wait()
            cl.wait()
            pltpu.sync_copy(rows_v, e_out.at[pl.ds(base, CH)])
            pltpu.sync_copy(lv_v, lv_out.at[pl.ds(base, CH)])

    return gather_k


def _relayout_body(tT_ref, o_ref):
    V = tT_ref.shape[2]
    C = V // 8
    for s in range(8):
        chunk = tT_ref[0, :, pl.ds(s * C, C)]
        o_ref[0, :, pl.ds(s * 16, 16)] = jnp.transpose(chunk)


def _relayout_table(tables):
    F, V, D = tables.shape
    tabT = jnp.transpose(tables, (0, 2, 1))
    return pl.pallas_call(
        _relayout_body,
        grid=(F,),
        in_specs=[pl.BlockSpec((1, D, V), lambda f: (f, 0, 0))],
        out_specs=pl.BlockSpec((1, V // 8, 128), lambda f: (f, 0, 0)),
        out_shape=jax.ShapeDtypeStruct((F, V // 8, 128), jnp.float32),
    )(tabT)


def _fm_body(e_ref, lv_ref, wblk_ref, bcat_ref, smat_ref, bias_ref, o_ref):
    E = e_ref[...]
    P = jnp.dot(E, wblk_ref[...],
                preferred_element_type=jnp.float32,
                precision=lax.Precision.HIGHEST) + bcat_ref[...]
    s = jnp.dot(P, smat_ref[...],
                preferred_element_type=jnp.float32,
                precision=lax.Precision.HIGHEST)
    ix = 0.5 * (jnp.sum(s * s, axis=1) - jnp.sum(P * P, axis=1))
    lin = jnp.sum(lv_ref[...], axis=1) + bias_ref[0]
    o_ref[...] = jax.nn.sigmoid(lin + ix)


def kernel(x, tables, W, b, lin_w, bias):
    F, V, D = tables.shape
    B = x.shape[0]
    N = B * F
    FD = F * D

    offs = (jnp.arange(F, dtype=x.dtype) * V)[None, :]
    flat_idx = (x + offs).reshape(-1)
    C = V // 8
    idx_e = (offs + (x % C) * 8 + x // C).reshape(-1)
    tab2 = _relayout_table(tables).reshape(F * V, D)
    lw1 = lin_w.reshape(-1)
    rf = jnp.arange(F)
    wblk = (jnp.zeros((F, D, F, D), jnp.float32)
            .at[rf, :, rf, :].set(W.astype(jnp.float32))
            .reshape(FD, FD))
    bcat = b.astype(jnp.float32).reshape(1, FD)
    smat = jnp.tile(jnp.eye(D, dtype=jnp.float32), (F, 1))
    bias1 = jnp.asarray(bias, jnp.float32).reshape(1)

    e_flat, lv = _build_sc_gather(N, D)(idx_e, flat_idx, tab2, lw1)
    e2 = e_flat.reshape(B, FD)
    lv2 = lv.reshape(B, F)

    BT = 2048
    out = pl.pallas_call(
        _fm_body,
        grid=(B // BT,),
        in_specs=[
            pl.BlockSpec((BT, FD), lambda i: (i, 0)),
            pl.BlockSpec((BT, F), lambda i: (i, 0)),
            pl.BlockSpec((FD, FD), lambda i: (0, 0)),
            pl.BlockSpec((1, FD), lambda i: (0, 0)),
            pl.BlockSpec((FD, D), lambda i: (0, 0)),
            pl.BlockSpec(memory_space=pltpu.SMEM),
        ],
        out_specs=pl.BlockSpec((BT,), lambda i: (i,)),
        out_shape=jax.ShapeDtypeStruct((B,), jnp.float32),
    )(e2, lv2, wblk, bcat, smat, bias1)
    return out
